# baseline (speedup 1.0000x reference)
import functools

import jax
import jax.numpy as jnp
from jax import lax
from jax.experimental import pallas as pl
from jax.experimental.pallas import tpu as pltpu
from jax.experimental.pallas import tpu_sc as plsc

N = 10000
E = 320000
D = 128
H = 256
G = 128

NC = 2
NS = 16
NW = NC * NS
K = 120
NBUF = 3
CPW = 84
KD = 120
CPWD = 84
E_PAD = NW * CPW * K
NACC = 10240
RPT = NACC // NS
ZSTEPS = [K] * (RPT // K) + ([RPT % K] if RPT % K else [])
ZSTEPSD = [KD] * (RPT // KD) + ([RPT % KD] if RPT % KD else [])

BN = 2000
NB = N // BN

_mesh = plsc.VectorSubcoreMesh(
    core_axis_name="c", subcore_axis_name="s", num_cores=NC, num_subcores=NS)


@functools.partial(
    pl.kernel,
    out_type=jax.ShapeDtypeStruct((NC * NACC, D), jnp.float32),
    mesh=_mesh,
    scratch_types=[
        [pltpu.VMEM((KD,), jnp.int32)] * 2,
        pltpu.VMEM((KD, D), jnp.float32),
        pltpu.VMEM((KD, D), jnp.float32),
        pltpu.VMEM_SHARED((NACC, D), jnp.float32),
        [pltpu.SemaphoreType.DMA] * 2,
    ],
)
def _sc_degree(dst_hbm, ones_hbm, zeros_hbm, out_hbm, dst_vs, ones_v, zero_v,
               acc_sh, ssems):
    c = lax.axis_index("c")
    s = lax.axis_index("s")
    wid = s * NC + c
    pltpu.sync_copy(ones_hbm, ones_v)
    pltpu.sync_copy(zeros_hbm, zero_v)
    zoff = 0
    for step in ZSTEPSD:
        pltpu.sync_copy(zero_v.at[pl.ds(0, step)],
                        acc_sh.at[pl.ds(s * RPT + zoff, step)])
        zoff += step
    plsc.subcore_barrier()
    base = wid * CPWD * KD

    def chunk(j, carry):
        for b in range(2):
            i = 2 * j + b

            @pl.when(j > 0)
            def _():
                pltpu.make_async_copy(zeros_hbm, zero_v, ssems[b]).wait()

            pltpu.sync_copy(dst_hbm.at[pl.ds(base + i * KD, KD)], dst_vs[b])
            pltpu.async_copy(ones_v, acc_sh.at[dst_vs[b]], ssems[b], add=True)
        return carry

    lax.fori_loop(0, CPWD // 2, chunk, 0)
    for b in range(2):
        pltpu.make_async_copy(zeros_hbm, zero_v, ssems[b]).wait()
    plsc.subcore_barrier()
    pltpu.sync_copy(acc_sh.at[pl.ds(s * RPT, RPT)],
                    out_hbm.at[pl.ds(c * NACC + s * RPT, RPT)])


@functools.partial(
    pl.kernel,
    out_type=jax.ShapeDtypeStruct((NC * NACC, D), jnp.float32),
    mesh=_mesh,
    scratch_types=[
        [[pltpu.VMEM((2, K), jnp.int32)] * NBUF] * 2,
        [pltpu.VMEM((K, D), jnp.float32)] * NBUF,
        pltpu.VMEM_SHARED((NACC, D), jnp.float32),
        [pltpu.SemaphoreType.DMA] * NBUF,
        [[pltpu.SemaphoreType.DMA] * NBUF] * 2,
        [[pltpu.SemaphoreType.DMA] * NBUF] * 2,
    ],
)
def _sc_scatter_rows(g_hbm, idx2_hbm, zeros_hbm, out_hbm,
                     idxs, rows, acc_sh, sems, ssems, isems):
    c = lax.axis_index("c")
    s = lax.axis_index("s")
    wid = s * NC + c
    pltpu.sync_copy(zeros_hbm, rows[0])
    zoff = 0
    for step in ZSTEPS:
        pltpu.sync_copy(rows[0].at[pl.ds(0, step)],
                        acc_sh.at[pl.ds(s * RPT + zoff, step)])
        zoff += step
    plsc.subcore_barrier()
    base = wid * CPW

    for b in range(NBUF):
        pltpu.async_copy(idx2_hbm.at[base + b], idxs[0][b], isems[0][b])

    def body(jj, carry):
        for p in range(2):
            np_ = 1 - p
            descs = []
            for b in range(NBUF):
                if p == 0:
                    @pl.when(jj > 0)
                    def _():
                        pltpu.make_async_copy(zeros_hbm, rows[b],
                                              ssems[np_][b]).wait()
                else:
                    pltpu.make_async_copy(zeros_hbm, rows[b],
                                          ssems[np_][b]).wait()
                pltpu.make_async_copy(idx2_hbm.at[base], idxs[p][b],
                                      isems[p][b]).wait()
                descs.append(pltpu.async_copy(g_hbm.at[idxs[p][b].at[0]],
                                              rows[b], sems[b]))
            for b in range(NBUF):
                lnext = lax.rem(jj * (2 * NBUF) + (p + 1) * NBUF + b, CPW)
                pltpu.async_copy(idx2_hbm.at[base + lnext], idxs[np_][b],
                                 isems[np_][b])
            for b in range(NBUF):
                descs[b].wait()
                pltpu.async_copy(rows[b], acc_sh.at[idxs[p][b].at[1]],
                                 ssems[p][b], add=True)
        return carry

    lax.fori_loop(0, CPW // (2 * NBUF), body, 0)
    for b in range(NBUF):
        pltpu.make_async_copy(zeros_hbm, rows[b], ssems[1][b]).wait()
        pltpu.make_async_copy(idx2_hbm.at[base], idxs[0][b],
                              isems[0][b]).wait()

    plsc.subcore_barrier()
    pltpu.sync_copy(acc_sh.at[pl.ds(s * RPT, RPT)],
                    out_hbm.at[pl.ds(c * NACC + s * RPT, RPT)])


def _tc0_body(x_ref, w_ref, degp_ref, g_ref, dinv_ref):
    deg = degp_ref[0, :, 0:1] + degp_ref[1, :, 0:1] + 1.0
    dinv = lax.rsqrt(deg)
    g_ref[...] = jnp.dot(x_ref[...], w_ref[...],
                         preferred_element_type=jnp.float32) * dinv
    dinv_ref[...] = jnp.broadcast_to(dinv, (BN, 16))


_tc0 = pl.pallas_call(
    _tc0_body,
    grid=(NB,),
    in_specs=[
        pl.BlockSpec((BN, D), lambda i: (i, 0)),
        pl.BlockSpec((D, D), lambda i: (0, 0)),
        pl.BlockSpec((NC, BN, D), lambda i: (0, i, 0)),
    ],
    out_specs=[
        pl.BlockSpec((BN, D), lambda i: (i, 0)),
        pl.BlockSpec((BN, 16), lambda i: (i, 0)),
    ],
    out_shape=[
        jax.ShapeDtypeStruct((N, D), jnp.float32),
        jax.ShapeDtypeStruct((N, 16), jnp.float32),
    ],
)


def _tc_layer_body(acc_ref, g_ref, dinv_ref, w_ref, b_ref, out_ref):
    dinv = dinv_ref[:, 0:1]
    h = jnp.maximum(dinv * (acc_ref[0] + acc_ref[1] + g_ref[...]) + b_ref[...], 0.0)
    out_ref[...] = jnp.dot(h, w_ref[...],
                           preferred_element_type=jnp.float32) * dinv


_tc_layer = pl.pallas_call(
    _tc_layer_body,
    grid=(NB,),
    in_specs=[
        pl.BlockSpec((NC, BN, D), lambda i: (0, i, 0)),
        pl.BlockSpec((BN, D), lambda i: (i, 0)),
        pl.BlockSpec((BN, 16), lambda i: (i, 0)),
        pl.BlockSpec((D, D), lambda i: (0, 0)),
        pl.BlockSpec((1, D), lambda i: (0, 0)),
    ],
    out_specs=pl.BlockSpec((BN, D), lambda i: (i, 0)),
    out_shape=jax.ShapeDtypeStruct((N, D), jnp.float32),
)


def _tc_final_body(acc_ref, g_ref, dinv_ref, b_ref, nw_ref, nb_ref, batch_ref,
                   f1w_ref, f1b_ref, f2w_ref, f2b_ref,
                   np_ref, sums_ref, cnt_ref, fea_ref):
    i = pl.program_id(0)
    dinv = dinv_ref[:, 0:1]
    h = jnp.maximum(dinv * (acc_ref[0] + acc_ref[1] + g_ref[...]) + b_ref[...], 0.0)
    np_ref[...] = jax.nn.sigmoid(
        jnp.dot(h, nw_ref[...], preferred_element_type=jnp.float32) + nb_ref[...])
    gid = lax.broadcasted_iota(jnp.int32, (BN, G), 1)
    mask = (batch_ref[...] == gid).astype(jnp.float32)
    psum = lax.dot_general(mask, h, (((0,), (0,)), ((), ())),
                           preferred_element_type=jnp.float32)
    pcnt = lax.dot_general(mask, jnp.ones((BN, 1), jnp.float32),
                           (((0,), (0,)), ((), ())),
                           preferred_element_type=jnp.float32)

    @pl.when(i == 0)
    def _():
        sums_ref[...] = psum
        cnt_ref[...] = pcnt

    @pl.when(i > 0)
    def _():
        sums_ref[...] += psum
        cnt_ref[...] += pcnt

    @pl.when(i == NB - 1)
    def _():
        fea = sums_ref[...] / jnp.maximum(cnt_ref[...], 1.0)
        fea = jnp.maximum(
            jnp.dot(fea, f1w_ref[...], preferred_element_type=jnp.float32)
            + f1b_ref[...], 0.0)
        fea_ref[...] = jax.nn.sigmoid(
            jnp.dot(fea, f2w_ref[...], preferred_element_type=jnp.float32)
            + f2b_ref[...])


_tc_final = pl.pallas_call(
    _tc_final_body,
    grid=(NB,),
    in_specs=[
        pl.BlockSpec((NC, BN, D), lambda i: (0, i, 0)),
        pl.BlockSpec((BN, D), lambda i: (i, 0)),
        pl.BlockSpec((BN, 16), lambda i: (i, 0)),
        pl.BlockSpec((1, D), lambda i: (0, 0)),
        pl.BlockSpec((D, 1), lambda i: (0, 0)),
        pl.BlockSpec((1, 1), lambda i: (0, 0)),
        pl.BlockSpec((BN, 1), lambda i: (i, 0)),
        pl.BlockSpec((D, H), lambda i: (0, 0)),
        pl.BlockSpec((1, H), lambda i: (0, 0)),
        pl.BlockSpec((H, D), lambda i: (0, 0)),
        pl.BlockSpec((1, D), lambda i: (0, 0)),
    ],
    out_specs=[
        pl.BlockSpec((BN, 1), lambda i: (i, 0)),
        pl.BlockSpec((G, D), lambda i: (0, 0)),
        pl.BlockSpec((G, 1), lambda i: (0, 0)),
        pl.BlockSpec((G, D), lambda i: (0, 0)),
    ],
    out_shape=[
        jax.ShapeDtypeStruct((N, 1), jnp.float32),
        jax.ShapeDtypeStruct((G, D), jnp.float32),
        jax.ShapeDtypeStruct((G, 1), jnp.float32),
        jax.ShapeDtypeStruct((G, D), jnp.float32),
    ],
)


def kernel(x, edge_index, batch, W0, b0, W1, b1, W2, b2,
           fea1_W, fea1_b, fea2_W, fea2_b, node_W, node_b):
    pad = E_PAD - E
    src_pad = jnp.arange(pad, dtype=jnp.int32) % N
    src_p = jnp.concatenate([edge_index[0], src_pad])
    dst_pad = N + jnp.arange(pad, dtype=jnp.int32) % (NACC - N)
    dst_p = jnp.concatenate([edge_index[1], dst_pad])
    idx2 = jnp.stack([src_p.reshape(E_PAD // K, K),
                      dst_p.reshape(E_PAD // K, K)], axis=1)
    onesKD = jnp.ones((KD, D), jnp.float32)
    zerosKD = jnp.zeros((KD, D), jnp.float32)
    zerosD = jnp.zeros((K, D), jnp.float32)

    degp = _sc_degree(dst_p, onesKD, zerosKD).reshape(NC, NACC, D)
    g, dinv = _tc0(x, W0, degp)

    acc = _sc_scatter_rows(g, idx2, zerosD).reshape(NC, NACC, D)
    g = _tc_layer(acc, g, dinv, W1, b0.reshape(1, D))

    acc = _sc_scatter_rows(g, idx2, zerosD).reshape(NC, NACC, D)
    g = _tc_layer(acc, g, dinv, W2, b1.reshape(1, D))

    acc = _sc_scatter_rows(g, idx2, zerosD).reshape(NC, NACC, D)
    node_prob, _, _, fea = _tc_final(
        acc, g, dinv, b2.reshape(1, D), node_W, node_b.reshape(1, 1),
        batch.reshape(N, 1), fea1_W, fea1_b.reshape(1, H),
        fea2_W, fea2_b.reshape(1, D))

    return (node_prob.reshape(N), fea)


# speedup vs baseline: 1.0197x; 1.0197x over previous
import functools

import jax
import jax.numpy as jnp
from jax import lax
from jax.experimental import pallas as pl
from jax.experimental.pallas import tpu as pltpu
from jax.experimental.pallas import tpu_sc as plsc

N = 10000
E = 320000
D = 128
H = 256
G = 128

NC = 2
NS = 16
NW = NC * NS
K = 120
NBUF = 3
CPW = 84
KD = 120
CPWD = 84
E_PAD = NW * CPW * K
NACC = 10240
RPT = NACC // NS
ZSTEPS = [K] * (RPT // K) + ([RPT % K] if RPT % K else [])
ZSTEPSD = [KD] * (RPT // KD) + ([RPT % KD] if RPT % KD else [])

BN = 2000
NB = N // BN

_mesh = plsc.VectorSubcoreMesh(
    core_axis_name="c", subcore_axis_name="s", num_cores=NC, num_subcores=NS)


@functools.partial(
    pl.kernel,
    out_type=jax.ShapeDtypeStruct((NC * NACC, D), jnp.float32),
    mesh=_mesh,
    scratch_types=[
        [pltpu.VMEM((KD,), jnp.int32)] * 2,
        pltpu.VMEM((KD, D), jnp.float32),
        pltpu.VMEM((KD, D), jnp.float32),
        pltpu.VMEM_SHARED((NACC, D), jnp.float32),
        [pltpu.SemaphoreType.DMA] * 2,
    ],
)
def _sc_degree(dst_hbm, ones_hbm, zeros_hbm, out_hbm, dst_vs, ones_v, zero_v,
               acc_sh, ssems):
    c = lax.axis_index("c")
    s = lax.axis_index("s")
    wid = s * NC + c
    pltpu.sync_copy(ones_hbm, ones_v)
    pltpu.sync_copy(zeros_hbm, zero_v)
    zoff = 0
    for step in ZSTEPSD:
        pltpu.sync_copy(zero_v.at[pl.ds(0, step)],
                        acc_sh.at[pl.ds(s * RPT + zoff, step)])
        zoff += step
    plsc.subcore_barrier()
    base = wid * CPWD * KD

    def chunk(j, carry):
        for b in range(2):
            i = 2 * j + b

            @pl.when(j > 0)
            def _():
                pltpu.make_async_copy(zeros_hbm, zero_v, ssems[b]).wait()

            pltpu.sync_copy(dst_hbm.at[pl.ds(base + i * KD, KD)], dst_vs[b])
            pltpu.async_copy(ones_v, acc_sh.at[dst_vs[b]], ssems[b], add=True)
        return carry

    lax.fori_loop(0, CPWD // 2, chunk, 0)
    for b in range(2):
        pltpu.make_async_copy(zeros_hbm, zero_v, ssems[b]).wait()
    plsc.subcore_barrier()
    pltpu.sync_copy(acc_sh.at[pl.ds(s * RPT, RPT)],
                    out_hbm.at[pl.ds(c * NACC + s * RPT, RPT)])


@functools.partial(
    pl.kernel,
    out_type=jax.ShapeDtypeStruct((NC * NACC, D), jnp.float32),
    mesh=_mesh,
    scratch_types=[
        [pltpu.VMEM((2, K), jnp.int32)] * NBUF,
        [pltpu.VMEM((K, D), jnp.float32)] * NBUF,
        pltpu.VMEM_SHARED((NACC, D), jnp.float32),
        [pltpu.SemaphoreType.DMA] * NBUF,
        [pltpu.SemaphoreType.DMA] * NBUF,
    ],
)
def _sc_scatter_rows(g_hbm, idx2_hbm, zeros_hbm, out_hbm,
                     idxs, rows, acc_sh, sems, ssems):
    c = lax.axis_index("c")
    s = lax.axis_index("s")
    wid = s * NC + c
    pltpu.sync_copy(zeros_hbm, rows[0])
    zoff = 0
    for step in ZSTEPS:
        pltpu.sync_copy(rows[0].at[pl.ds(0, step)],
                        acc_sh.at[pl.ds(s * RPT + zoff, step)])
        zoff += step
    plsc.subcore_barrier()
    base = wid * CPW

    def body(j, carry):
        descs = []
        for b in range(NBUF):
            i = base + NBUF * j + b

            @pl.when(j > 0)
            def _():
                pltpu.make_async_copy(zeros_hbm, rows[b], ssems[b]).wait()

            pltpu.sync_copy(idx2_hbm.at[i], idxs[b])
            descs.append(
                pltpu.async_copy(g_hbm.at[idxs[b].at[0]], rows[b], sems[b]))
        for b in range(NBUF):
            descs[b].wait()
            pltpu.async_copy(rows[b], acc_sh.at[idxs[b].at[1]], ssems[b],
                             add=True)
        return carry

    lax.fori_loop(0, CPW // NBUF, body, 0)
    for b in range(NBUF):
        pltpu.make_async_copy(zeros_hbm, rows[b], ssems[b]).wait()

    plsc.subcore_barrier()
    pltpu.sync_copy(acc_sh.at[pl.ds(s * RPT, RPT)],
                    out_hbm.at[pl.ds(c * NACC + s * RPT, RPT)])


def _tc0_body(x_ref, w_ref, degp_ref, g_ref, dinv_ref):
    deg = degp_ref[0, :, 0:1] + degp_ref[1, :, 0:1] + 1.0
    dinv = lax.rsqrt(deg)
    g_ref[...] = jnp.dot(x_ref[...], w_ref[...],
                         preferred_element_type=jnp.float32) * dinv
    dinv_ref[...] = jnp.broadcast_to(dinv, (BN, 16))


_tc0 = pl.pallas_call(
    _tc0_body,
    grid=(NB,),
    in_specs=[
        pl.BlockSpec((BN, D), lambda i: (i, 0)),
        pl.BlockSpec((D, D), lambda i: (0, 0)),
        pl.BlockSpec((NC, BN, D), lambda i: (0, i, 0)),
    ],
    out_specs=[
        pl.BlockSpec((BN, D), lambda i: (i, 0)),
        pl.BlockSpec((BN, 16), lambda i: (i, 0)),
    ],
    out_shape=[
        jax.ShapeDtypeStruct((N, D), jnp.float32),
        jax.ShapeDtypeStruct((N, 16), jnp.float32),
    ],
)


def _tc_layer_body(acc_ref, g_ref, dinv_ref, w_ref, b_ref, out_ref):
    dinv = dinv_ref[:, 0:1]
    h = jnp.maximum(dinv * (acc_ref[0] + acc_ref[1] + g_ref[...]) + b_ref[...], 0.0)
    out_ref[...] = jnp.dot(h, w_ref[...],
                           preferred_element_type=jnp.float32) * dinv


_tc_layer = pl.pallas_call(
    _tc_layer_body,
    grid=(NB,),
    in_specs=[
        pl.BlockSpec((NC, BN, D), lambda i: (0, i, 0)),
        pl.BlockSpec((BN, D), lambda i: (i, 0)),
        pl.BlockSpec((BN, 16), lambda i: (i, 0)),
        pl.BlockSpec((D, D), lambda i: (0, 0)),
        pl.BlockSpec((1, D), lambda i: (0, 0)),
    ],
    out_specs=pl.BlockSpec((BN, D), lambda i: (i, 0)),
    out_shape=jax.ShapeDtypeStruct((N, D), jnp.float32),
)


def _tc_final_body(acc_ref, g_ref, dinv_ref, b_ref, nw_ref, nb_ref, batch_ref,
                   f1w_ref, f1b_ref, f2w_ref, f2b_ref,
                   np_ref, sums_ref, cnt_ref, fea_ref):
    i = pl.program_id(0)
    dinv = dinv_ref[:, 0:1]
    h = jnp.maximum(dinv * (acc_ref[0] + acc_ref[1] + g_ref[...]) + b_ref[...], 0.0)
    np_ref[...] = jax.nn.sigmoid(
        jnp.dot(h, nw_ref[...], preferred_element_type=jnp.float32) + nb_ref[...])
    gid = lax.broadcasted_iota(jnp.int32, (BN, G), 1)
    mask = (batch_ref[...] == gid).astype(jnp.float32)
    psum = lax.dot_general(mask, h, (((0,), (0,)), ((), ())),
                           preferred_element_type=jnp.float32)
    pcnt = lax.dot_general(mask, jnp.ones((BN, 1), jnp.float32),
                           (((0,), (0,)), ((), ())),
                           preferred_element_type=jnp.float32)

    @pl.when(i == 0)
    def _():
        sums_ref[...] = psum
        cnt_ref[...] = pcnt

    @pl.when(i > 0)
    def _():
        sums_ref[...] += psum
        cnt_ref[...] += pcnt

    @pl.when(i == NB - 1)
    def _():
        fea = sums_ref[...] / jnp.maximum(cnt_ref[...], 1.0)
        fea = jnp.maximum(
            jnp.dot(fea, f1w_ref[...], preferred_element_type=jnp.float32)
            + f1b_ref[...], 0.0)
        fea_ref[...] = jax.nn.sigmoid(
            jnp.dot(fea, f2w_ref[...], preferred_element_type=jnp.float32)
            + f2b_ref[...])


_tc_final = pl.pallas_call(
    _tc_final_body,
    grid=(NB,),
    in_specs=[
        pl.BlockSpec((NC, BN, D), lambda i: (0, i, 0)),
        pl.BlockSpec((BN, D), lambda i: (i, 0)),
        pl.BlockSpec((BN, 16), lambda i: (i, 0)),
        pl.BlockSpec((1, D), lambda i: (0, 0)),
        pl.BlockSpec((D, 1), lambda i: (0, 0)),
        pl.BlockSpec((1, 1), lambda i: (0, 0)),
        pl.BlockSpec((BN, 1), lambda i: (i, 0)),
        pl.BlockSpec((D, H), lambda i: (0, 0)),
        pl.BlockSpec((1, H), lambda i: (0, 0)),
        pl.BlockSpec((H, D), lambda i: (0, 0)),
        pl.BlockSpec((1, D), lambda i: (0, 0)),
    ],
    out_specs=[
        pl.BlockSpec((BN, 1), lambda i: (i, 0)),
        pl.BlockSpec((G, D), lambda i: (0, 0)),
        pl.BlockSpec((G, 1), lambda i: (0, 0)),
        pl.BlockSpec((G, D), lambda i: (0, 0)),
    ],
    out_shape=[
        jax.ShapeDtypeStruct((N, 1), jnp.float32),
        jax.ShapeDtypeStruct((G, D), jnp.float32),
        jax.ShapeDtypeStruct((G, 1), jnp.float32),
        jax.ShapeDtypeStruct((G, D), jnp.float32),
    ],
)


def kernel(x, edge_index, batch, W0, b0, W1, b1, W2, b2,
           fea1_W, fea1_b, fea2_W, fea2_b, node_W, node_b):
    pad = E_PAD - E
    src_pad = jnp.arange(pad, dtype=jnp.int32) % N
    src_p = jnp.concatenate([edge_index[0], src_pad])
    dst_pad = N + jnp.arange(pad, dtype=jnp.int32) % (NACC - N)
    dst_p = jnp.concatenate([edge_index[1], dst_pad])
    idx2 = jnp.stack([src_p.reshape(E_PAD // K, K),
                      dst_p.reshape(E_PAD // K, K)], axis=1)
    onesKD = jnp.ones((KD, D), jnp.float32)
    zerosKD = jnp.zeros((KD, D), jnp.float32)
    zerosD = jnp.zeros((K, D), jnp.float32)

    degp = _sc_degree(dst_p, onesKD, zerosKD).reshape(NC, NACC, D)
    g, dinv = _tc0(x, W0, degp)

    acc = _sc_scatter_rows(g, idx2, zerosD).reshape(NC, NACC, D)
    g = _tc_layer(acc, g, dinv, W1, b0.reshape(1, D))

    acc = _sc_scatter_rows(g, idx2, zerosD).reshape(NC, NACC, D)
    g = _tc_layer(acc, g, dinv, W2, b1.reshape(1, D))

    acc = _sc_scatter_rows(g, idx2, zerosD).reshape(NC, NACC, D)
    node_prob, _, _, fea = _tc_final(
        acc, g, dinv, b2.reshape(1, D), node_W, node_b.reshape(1, 1),
        batch.reshape(N, 1), fea1_W, fea1_b.reshape(1, H),
        fea2_W, fea2_b.reshape(1, D))

    return (node_prob.reshape(N), fea)
